# baseline (device time: 71797 ns/iter reference)
import jax
import jax.numpy as jnp
from jax import lax
from jax.experimental import pallas as pl
from jax.experimental.pallas import tpu as pltpu

N_DEV = 8
B = 2
S = 256
D = 512
HQ = 4
DH = 64
HD = HQ * DH
S_FULL = N_DEV * S


def kernel(x, Wq, K_ext, V_ext, Wo):
    xb = x.astype(jnp.bfloat16)
    wq = Wq.astype(jnp.bfloat16)
    wo = Wo.astype(jnp.bfloat16)
    kb = K_ext.reshape(B, S, HD).astype(jnp.bfloat16)
    vb = V_ext.reshape(B, S, HD).astype(jnp.bfloat16)

    def body(x_ref, wq_ref, k_ref, v_ref, wo_ref, out_ref,
             kv_full, send_sems, recv_sems):
        my = lax.axis_index("i")
        left = lax.rem(my - 1 + N_DEV, N_DEV)
        right = lax.rem(my + 1, N_DEV)

        barrier_sem = pltpu.get_barrier_semaphore()
        for nbr in (left, right):
            pl.semaphore_signal(
                barrier_sem, inc=1,
                device_id=(nbr,), device_id_type=pl.DeviceIdType.MESH,
            )
        pl.semaphore_wait(barrier_sem, 2)

        my_off = my * S
        kv_full[:, pl.ds(my_off, S), 0:HD] = k_ref[...]
        kv_full[:, pl.ds(my_off, S), HD:2 * HD] = v_ref[...]

        for h in range(N_DEV - 1):
            chunk = lax.rem(my - h + N_DEV, N_DEV)
            off = chunk * S
            rdma = pltpu.make_async_remote_copy(
                src_ref=kv_full.at[:, pl.ds(off, S), :],
                dst_ref=kv_full.at[:, pl.ds(off, S), :],
                send_sem=send_sems.at[h],
                recv_sem=recv_sems.at[h],
                device_id=(right,),
                device_id_type=pl.DeviceIdType.MESH,
            )
            rdma.start()
            rdma.wait()

        wq_v = wq_ref[...]
        wo_v = wo_ref[...]

        iq = lax.broadcasted_iota(jnp.int32, (S, S_FULL), 0)
        jk = lax.broadcasted_iota(jnp.int32, (S, S_FULL), 1)
        qblk = my * (S // 64) + iq // 64
        kblk = jk // 64
        mask = (qblk == kblk) | (kblk == 0) | (lax.rem(qblk + kblk, 3) == 0)

        for b in range(B):
            q_b = jnp.dot(x_ref[b], wq_v,
                          preferred_element_type=jnp.float32)
            q_b = q_b.astype(jnp.bfloat16)
            ctx_heads = []
            for h in range(HQ):
                q_h = q_b[:, h * DH:(h + 1) * DH]
                k_h = kv_full[b, :, h * DH:(h + 1) * DH]
                v_h = kv_full[b, :, HD + h * DH:HD + (h + 1) * DH]
                scores = lax.dot_general(
                    q_h, k_h,
                    dimension_numbers=(((1,), (1,)), ((), ())),
                    preferred_element_type=jnp.float32,
                ) * 0.125
                scores = jnp.where(mask, scores, -1e9)
                m = jnp.max(scores, axis=1, keepdims=True)
                w = jnp.exp(scores - m)
                w = w / jnp.sum(w, axis=1, keepdims=True)
                ctx = jnp.dot(w.astype(jnp.bfloat16), v_h,
                              preferred_element_type=jnp.float32)
                ctx_heads.append(ctx.astype(jnp.bfloat16))
            ctx_b = jnp.concatenate(ctx_heads, axis=1)
            out_ref[b] = jnp.dot(ctx_b, wo_v,
                                 preferred_element_type=jnp.float32)

    return pl.pallas_call(
        body,
        out_shape=jax.ShapeDtypeStruct((B, S, D), jnp.float32),
        in_specs=[pl.BlockSpec(memory_space=pltpu.VMEM)] * 5,
        out_specs=pl.BlockSpec(memory_space=pltpu.VMEM),
        scratch_shapes=[
            pltpu.VMEM((B, S_FULL, 2 * HD), jnp.bfloat16),
            pltpu.SemaphoreType.DMA((N_DEV - 1,)),
            pltpu.SemaphoreType.DMA((N_DEV - 1,)),
        ],
        compiler_params=pltpu.CompilerParams(collective_id=0),
    )(xb, wq, kb, vb, wo)


# device time: 44215 ns/iter; 1.6238x vs baseline; 1.6238x over previous
import jax
import jax.numpy as jnp
from jax import lax
from jax.experimental import pallas as pl
from jax.experimental.pallas import tpu as pltpu

N_DEV = 8
B = 2
S = 256
D = 512
HQ = 4
DH = 64
HD = HQ * DH
S_FULL = N_DEV * S
HOPS_R = 4
HOPS_L = 3


def kernel(x, Wq, K_ext, V_ext, Wo):
    xb = x.astype(jnp.bfloat16)
    wq = Wq.astype(jnp.bfloat16)
    wo = Wo.astype(jnp.bfloat16)
    kb = K_ext.reshape(B, S, HD).astype(jnp.bfloat16)
    vb = V_ext.reshape(B, S, HD).astype(jnp.bfloat16)

    def body(x_ref, wq_ref, k_ref, v_ref, wo_ref, out_ref,
             kv_full, ss_r, rs_r, ss_l, rs_l):
        my = lax.axis_index("i")
        left = lax.rem(my - 1 + N_DEV, N_DEV)
        right = lax.rem(my + 1, N_DEV)

        barrier_sem = pltpu.get_barrier_semaphore()
        for nbr in (left, right):
            pl.semaphore_signal(
                barrier_sem, inc=1,
                device_id=(nbr,), device_id_type=pl.DeviceIdType.MESH,
            )
        pl.semaphore_wait(barrier_sem, 2)

        kv_full[:, pl.ds(my * S, S), 0:HD] = k_ref[...]
        kv_full[:, pl.ds(my * S, S), HD:2 * HD] = v_ref[...]

        def mk(h, chunk, dev, ss, rs):
            off = chunk * S
            return pltpu.make_async_remote_copy(
                src_ref=kv_full.at[:, pl.ds(off, S), :],
                dst_ref=kv_full.at[:, pl.ds(off, S), :],
                send_sem=ss.at[h],
                recv_sem=rs.at[h],
                device_id=(dev,),
                device_id_type=pl.DeviceIdType.MESH,
            )

        def mk_r(h):
            return mk(h, lax.rem(my - h + N_DEV, N_DEV), right, ss_r, rs_r)

        def mk_l(h):
            return mk(h, lax.rem(my + h, N_DEV), left, ss_l, rs_l)

        rdma_r = [mk_r(0)]
        rdma_l = [mk_l(0)]
        rdma_r[0].start()
        rdma_l[0].start()

        q_vals = []
        for b in range(B):
            q_b = jnp.dot(x_ref[b], wq_ref[...],
                          preferred_element_type=jnp.float32)
            q_vals.append(q_b.astype(jnp.bfloat16))

        iq = lax.broadcasted_iota(jnp.int32, (S, S), 0)
        jk = lax.broadcasted_iota(jnp.int32, (S, S), 1)
        qblk = my * (S // 64) + iq // 64
        kloc = jk // 64

        scores = {(b, h): [] for b in range(B) for h in range(HQ)}
        v_chunks = {b: [] for b in range(B)}

        def process_chunk(c):
            kblk = c * (S // 64) + kloc
            m = (qblk == kblk) | (kblk == 0) | (lax.rem(qblk + kblk, 3) == 0)
            for b in range(B):
                k_c = kv_full[b, pl.ds(c * S, S), 0:HD]
                v_c = kv_full[b, pl.ds(c * S, S), HD:2 * HD]
                v_chunks[b].append(v_c)
                for h in range(HQ):
                    q_h = q_vals[b][:, h * DH:(h + 1) * DH]
                    k_h = k_c[:, h * DH:(h + 1) * DH]
                    s_c = lax.dot_general(
                        q_h, k_h,
                        dimension_numbers=(((1,), (1,)), ((), ())),
                        preferred_element_type=jnp.float32,
                    ) * 0.125
                    scores[(b, h)].append(jnp.where(m, s_c, -1e9))

        process_chunk(my)

        for h in range(HOPS_R):
            rdma_r[h].wait()
            if h + 1 < HOPS_R:
                rdma_r.append(mk_r(h + 1))
                rdma_r[h + 1].start()
            if h < HOPS_L:
                rdma_l[h].wait()
                if h + 1 < HOPS_L:
                    rdma_l.append(mk_l(h + 1))
                    rdma_l[h + 1].start()
            process_chunk(lax.rem(my - 1 - h + N_DEV, N_DEV))
            if h < HOPS_L:
                process_chunk(lax.rem(my + 1 + h, N_DEV))

        for b in range(B):
            ctx_heads = []
            for h in range(HQ):
                sc = jnp.concatenate(scores[(b, h)], axis=1)
                mx = jnp.max(sc, axis=1, keepdims=True)
                w = jnp.exp(sc - mx)
                w = (w / jnp.sum(w, axis=1, keepdims=True)).astype(jnp.bfloat16)
                v_arr = jnp.concatenate(
                    [v[:, h * DH:(h + 1) * DH] for v in v_chunks[b]], axis=0
                )
                ctx = jnp.dot(w, v_arr, preferred_element_type=jnp.float32)
                ctx_heads.append(ctx.astype(jnp.bfloat16))
            ctx_b = jnp.concatenate(ctx_heads, axis=1)
            out_ref[b] = jnp.dot(ctx_b, wo_ref[...],
                                 preferred_element_type=jnp.float32)

    return pl.pallas_call(
        body,
        out_shape=jax.ShapeDtypeStruct((B, S, D), jnp.float32),
        in_specs=[pl.BlockSpec(memory_space=pltpu.VMEM)] * 5,
        out_specs=pl.BlockSpec(memory_space=pltpu.VMEM),
        scratch_shapes=[
            pltpu.VMEM((B, S_FULL, 2 * HD), jnp.bfloat16),
            pltpu.SemaphoreType.DMA((HOPS_R,)),
            pltpu.SemaphoreType.DMA((HOPS_R,)),
            pltpu.SemaphoreType.DMA((HOPS_L,)),
            pltpu.SemaphoreType.DMA((HOPS_L,)),
        ],
        compiler_params=pltpu.CompilerParams(collective_id=0),
    )(xb, wq, kb, vb, wo)


# device time: 39955 ns/iter; 1.7969x vs baseline; 1.1066x over previous
import jax
import jax.numpy as jnp
from jax import lax
from jax.experimental import pallas as pl
from jax.experimental.pallas import tpu as pltpu

N_DEV = 8
B = 2
S = 256
SH = S // 2
D = 512
HQ = 4
DH = 64
HD = HQ * DH
S_FULL = N_DEV * S
HOPS = 4


def kernel(x, Wq, K_ext, V_ext, Wo):
    xb = x.astype(jnp.bfloat16)
    wq = Wq.astype(jnp.bfloat16)
    wo = Wo.astype(jnp.bfloat16)
    kb = K_ext.reshape(B, S, HD).astype(jnp.bfloat16)
    vb = V_ext.reshape(B, S, HD).astype(jnp.bfloat16)

    def body(x_ref, wq_ref, k_ref, v_ref, wo_ref, out_ref,
             kv_full, ss_r, rs_r, ss_l, rs_l):
        my = lax.axis_index("i")
        left = lax.rem(my - 1 + N_DEV, N_DEV)
        right = lax.rem(my + 1, N_DEV)

        barrier_sem = pltpu.get_barrier_semaphore()
        for nbr in (left, right):
            pl.semaphore_signal(
                barrier_sem, inc=1,
                device_id=(nbr,), device_id_type=pl.DeviceIdType.MESH,
            )
        pl.semaphore_wait(barrier_sem, 2)

        kv_full[:, pl.ds(my * S, S), 0:HD] = k_ref[...]
        kv_full[:, pl.ds(my * S, S), HD:2 * HD] = v_ref[...]

        def mk(h, chunk, row_off, rows, dev, ss, rs):
            off = chunk * S + row_off
            return pltpu.make_async_remote_copy(
                src_ref=kv_full.at[:, pl.ds(off, rows), :],
                dst_ref=kv_full.at[:, pl.ds(off, rows), :],
                send_sem=ss.at[h],
                recv_sem=rs.at[h],
                device_id=(dev,),
                device_id_type=pl.DeviceIdType.MESH,
            )

        def mk_r(h):
            c = lax.rem(my - h + N_DEV, N_DEV)
            if h < HOPS - 1:
                return mk(h, c, 0, S, right, ss_r, rs_r)
            return mk(h, c, 0, SH, right, ss_r, rs_r)

        def mk_l(h):
            c = lax.rem(my + h, N_DEV)
            if h < HOPS - 1:
                return mk(h, c, 0, S, left, ss_l, rs_l)
            return mk(h, c, SH, SH, left, ss_l, rs_l)

        rdma_r = [mk_r(0)]
        rdma_l = [mk_l(0)]
        rdma_r[0].start()
        rdma_l[0].start()

        q_vals = []
        for b in range(B):
            q_b = jnp.dot(x_ref[b], wq_ref[...],
                          preferred_element_type=jnp.float32)
            q_vals.append(q_b.astype(jnp.bfloat16))

        iq = lax.broadcasted_iota(jnp.int32, (S, S), 0)
        jk = lax.broadcasted_iota(jnp.int32, (S, S), 1)
        qblk = my * (S // 64) + iq // 64
        kloc = jk // 64

        st = {}

        def process_chunk(c, first):
            kblk = c * (S // 64) + kloc
            msk = (qblk == kblk) | (kblk == 0) | (lax.rem(qblk + kblk, 3) == 0)
            for b in range(B):
                k_c = kv_full[b, pl.ds(c * S, S), 0:HD]
                v_c = kv_full[b, pl.ds(c * S, S), HD:2 * HD]
                for h in range(HQ):
                    q_h = q_vals[b][:, h * DH:(h + 1) * DH]
                    k_h = k_c[:, h * DH:(h + 1) * DH]
                    v_h = v_c[:, h * DH:(h + 1) * DH]
                    s_c = lax.dot_general(
                        q_h, k_h,
                        dimension_numbers=(((1,), (1,)), ((), ())),
                        preferred_element_type=jnp.float32,
                    ) * 0.125
                    s_c = jnp.where(msk, s_c, -1e9)
                    m_c = jnp.max(s_c, axis=1, keepdims=True)
                    if first:
                        p = jnp.exp(s_c - m_c)
                        st[(b, h)] = (
                            m_c,
                            jnp.sum(p, axis=1, keepdims=True),
                            jnp.dot(p.astype(jnp.bfloat16), v_h,
                                    preferred_element_type=jnp.float32),
                        )
                    else:
                        m, l, acc = st[(b, h)]
                        m_new = jnp.maximum(m, m_c)
                        alpha = jnp.exp(m - m_new)
                        p = jnp.exp(s_c - m_new)
                        st[(b, h)] = (
                            m_new,
                            l * alpha + jnp.sum(p, axis=1, keepdims=True),
                            acc * alpha
                            + jnp.dot(p.astype(jnp.bfloat16), v_h,
                                      preferred_element_type=jnp.float32),
                        )

        process_chunk(my, first=True)

        for h in range(HOPS):
            rdma_r[h].wait()
            rdma_l[h].wait()
            if h + 1 < HOPS:
                rdma_r.append(mk_r(h + 1))
                rdma_l.append(mk_l(h + 1))
                rdma_r[h + 1].start()
                rdma_l[h + 1].start()
                process_chunk(lax.rem(my - 1 - h + N_DEV, N_DEV), False)
                process_chunk(lax.rem(my + 1 + h, N_DEV), False)
            else:
                process_chunk(lax.rem(my + HOPS, N_DEV), False)

        for b in range(B):
            ctx_heads = []
            for h in range(HQ):
                m, l, acc = st[(b, h)]
                ctx_heads.append((acc / l).astype(jnp.bfloat16))
            ctx_b = jnp.concatenate(ctx_heads, axis=1)
            out_ref[b] = jnp.dot(ctx_b, wo_ref[...],
                                 preferred_element_type=jnp.float32)

    return pl.pallas_call(
        body,
        out_shape=jax.ShapeDtypeStruct((B, S, D), jnp.float32),
        in_specs=[pl.BlockSpec(memory_space=pltpu.VMEM)] * 5,
        out_specs=pl.BlockSpec(memory_space=pltpu.VMEM),
        scratch_shapes=[
            pltpu.VMEM((B, S_FULL, 2 * HD), jnp.bfloat16),
            pltpu.SemaphoreType.DMA((HOPS,)),
            pltpu.SemaphoreType.DMA((HOPS,)),
            pltpu.SemaphoreType.DMA((HOPS,)),
            pltpu.SemaphoreType.DMA((HOPS,)),
        ],
        compiler_params=pltpu.CompilerParams(collective_id=0),
    )(xb, wq, kb, vb, wo)


# device time: 29143 ns/iter; 2.4636x vs baseline; 1.3710x over previous
import jax
import jax.numpy as jnp
from jax import lax
from jax.experimental import pallas as pl
from jax.experimental.pallas import tpu as pltpu

N_DEV = 8
B = 2
S = 256
D = 512
HQ = 4
DH = 64
HD = HQ * DH


def _gray(p):
    return jnp.where(p < 4, p, 11 - p)


def kernel(x, Wq, K_ext, V_ext, Wo):
    xb = x.astype(jnp.bfloat16)
    wq = Wq.astype(jnp.bfloat16)
    wo = Wo.astype(jnp.bfloat16)
    kf = K_ext.reshape(B, S, HD)
    vf = V_ext.reshape(B, S, HD)

    def body(x_ref, wq_ref, k_ref, v_ref, wo_ref, out_ref,
             kv_i8, scl, ss_r, rs_r, ss_l, rs_l, ss_c, rs_c,
             ss_r2, rs_r2, ss_l2, rs_l2, ss_c2, rs_c2):
        my = lax.axis_index("i")
        g = _gray(my)
        even = lax.rem(g, 2) == 0

        def cyc(d):
            return _gray(lax.rem(g + d + 2 * N_DEV, N_DEV))

        def sel(a, b):
            return jnp.where(even, a, b)

        right = cyc(1)
        left = cyc(-1)
        chord = sel(cyc(3), cyc(-3))

        for ref, which in ((k_ref, 0), (v_ref, 1)):
            val = ref[...]
            mx = jnp.maximum(
                jnp.max(jnp.abs(val), axis=-1, keepdims=True), 1e-9)
            q = jnp.clip(jnp.round(val * (127.0 / mx)), -127, 127)
            kv_i8[pl.ds(my, 1), :, :, which * HD:(which + 1) * HD] = (
                q.astype(jnp.int8)[None])
            scl[pl.ds(my, 1), which] = mx[..., 0][None]

        barrier_sem = pltpu.get_barrier_semaphore()
        for nbr in (left, right, chord):
            pl.semaphore_signal(
                barrier_sem, inc=1,
                device_id=(nbr,), device_id_type=pl.DeviceIdType.MESH,
            )
        pl.semaphore_wait(barrier_sem, 3)

        def mk_pair(h, chunk, dev, ss, rs, ss2, rs2):
            data = pltpu.make_async_remote_copy(
                src_ref=kv_i8.at[chunk],
                dst_ref=kv_i8.at[chunk],
                send_sem=ss.at[h], recv_sem=rs.at[h],
                device_id=(dev,), device_id_type=pl.DeviceIdType.MESH,
            )
            scale = pltpu.make_async_remote_copy(
                src_ref=scl.at[chunk],
                dst_ref=scl.at[chunk],
                send_sem=ss2.at[h], recv_sem=rs2.at[h],
                device_id=(dev,), device_id_type=pl.DeviceIdType.MESH,
            )
            return (data, scale)

        def start(pair):
            pair[0].start()
            pair[1].start()
            return pair

        def wait(pair):
            pair[0].wait()
            pair[1].wait()

        r_send = [my, cyc(-1)]
        l_send = [my, cyc(1)]
        c_send = [my, sel(cyc(-1), cyc(1)), sel(cyc(-2), cyc(2))]
        r_recv = [cyc(-1), cyc(-2)]
        l_recv = [cyc(1), cyc(2)]
        c_recv = [sel(cyc(3), cyc(-3)), cyc(4), sel(cyc(-3), cyc(3))]

        mk_r = lambda h: mk_pair(h, r_send[h], right, ss_r, rs_r, ss_r2, rs_r2)
        mk_l = lambda h: mk_pair(h, l_send[h], left, ss_l, rs_l, ss_l2, rs_l2)
        mk_c = lambda h: mk_pair(h, c_send[h], chord, ss_c, rs_c, ss_c2, rs_c2)

        rdma_r = [start(mk_r(0))]
        rdma_l = [start(mk_l(0))]
        rdma_c = [start(mk_c(0))]

        q_vals = []
        for b in range(B):
            q_b = jnp.dot(x_ref[b], wq_ref[...],
                          preferred_element_type=jnp.float32)
            q_vals.append(q_b.astype(jnp.bfloat16))

        iq = lax.broadcasted_iota(jnp.int32, (S, S), 0)
        jk = lax.broadcasted_iota(jnp.int32, (S, S), 1)
        qblk = my * (S // 64) + iq // 64
        kloc = jk // 64

        st = {}

        def process_chunk(c, first):
            kblk = c * (S // 64) + kloc
            msk = (qblk == kblk) | (kblk == 0) | (lax.rem(qblk + kblk, 3) == 0)
            for b in range(B):
                kv_c = kv_i8[pl.ds(c, 1), b]
                k_c = kv_c[0, :, 0:HD].astype(jnp.bfloat16)
                v_c = kv_c[0, :, HD:2 * HD].astype(jnp.bfloat16)
                sk = scl[pl.ds(c, 1), 0, b] * (0.125 / 127.0)
                sv = scl[pl.ds(c, 1), 1, b] * (1.0 / 127.0)
                for h in range(HQ):
                    q_h = q_vals[b][:, h * DH:(h + 1) * DH]
                    k_h = k_c[:, h * DH:(h + 1) * DH]
                    v_h = v_c[:, h * DH:(h + 1) * DH]
                    s_c = lax.dot_general(
                        q_h, k_h,
                        dimension_numbers=(((1,), (1,)), ((), ())),
                        preferred_element_type=jnp.float32,
                    ) * sk
                    s_c = jnp.where(msk, s_c, -1e9)
                    m_c = jnp.max(s_c, axis=1, keepdims=True)
                    if first:
                        p = jnp.exp(s_c - m_c)
                        st[(b, h)] = (
                            m_c,
                            jnp.sum(p, axis=1, keepdims=True),
                            jnp.dot((p * sv).astype(jnp.bfloat16), v_h,
                                    preferred_element_type=jnp.float32),
                        )
                    else:
                        m, l, acc = st[(b, h)]
                        m_new = jnp.maximum(m, m_c)
                        alpha = jnp.exp(m - m_new)
                        p = jnp.exp(s_c - m_new)
                        st[(b, h)] = (
                            m_new,
                            l * alpha + jnp.sum(p, axis=1, keepdims=True),
                            acc * alpha
                            + jnp.dot((p * sv).astype(jnp.bfloat16), v_h,
                                      preferred_element_type=jnp.float32),
                        )

        process_chunk(my, first=True)

        wait(rdma_r[0])
        wait(rdma_l[0])
        wait(rdma_c[0])
        rdma_r.append(start(mk_r(1)))
        rdma_l.append(start(mk_l(1)))
        rdma_c.append(start(mk_c(1)))
        process_chunk(r_recv[0], False)
        process_chunk(l_recv[0], False)
        process_chunk(c_recv[0], False)

        wait(rdma_r[1])
        wait(rdma_l[1])
        wait(rdma_c[1])
        rdma_c.append(start(mk_c(2)))
        process_chunk(r_recv[1], False)
        process_chunk(l_recv[1], False)
        process_chunk(c_recv[1], False)

        wait(rdma_c[2])
        process_chunk(c_recv[2], False)

        for b in range(B):
            ctx_heads = []
            for h in range(HQ):
                m, l, acc = st[(b, h)]
                ctx_heads.append((acc / l).astype(jnp.bfloat16))
            ctx_b = jnp.concatenate(ctx_heads, axis=1)
            out_ref[b] = jnp.dot(ctx_b, wo_ref[...],
                                 preferred_element_type=jnp.float32)

    return pl.pallas_call(
        body,
        out_shape=jax.ShapeDtypeStruct((B, S, D), jnp.float32),
        in_specs=[pl.BlockSpec(memory_space=pltpu.VMEM)] * 5,
        out_specs=pl.BlockSpec(memory_space=pltpu.VMEM),
        scratch_shapes=[
            pltpu.VMEM((N_DEV, B, S, 2 * HD), jnp.int8),
            pltpu.VMEM((N_DEV, 2, B, S), jnp.float32),
            pltpu.SemaphoreType.DMA((2,)),
            pltpu.SemaphoreType.DMA((2,)),
            pltpu.SemaphoreType.DMA((2,)),
            pltpu.SemaphoreType.DMA((2,)),
            pltpu.SemaphoreType.DMA((3,)),
            pltpu.SemaphoreType.DMA((3,)),
            pltpu.SemaphoreType.DMA((2,)),
            pltpu.SemaphoreType.DMA((2,)),
            pltpu.SemaphoreType.DMA((2,)),
            pltpu.SemaphoreType.DMA((2,)),
            pltpu.SemaphoreType.DMA((3,)),
            pltpu.SemaphoreType.DMA((3,)),
        ],
        compiler_params=pltpu.CompilerParams(collective_id=0),
    )(xb, wq, kf, vf, wo)


# device time: 28465 ns/iter; 2.5223x vs baseline; 1.0238x over previous
import jax
import jax.numpy as jnp
from jax import lax
from jax.experimental import pallas as pl
from jax.experimental.pallas import tpu as pltpu

N_DEV = 8
B = 2
S = 256
D = 512
HQ = 4
DH = 64
HD = HQ * DH


def _gray(p):
    return jnp.where(p < 4, p, 11 - p)


def kernel(x, Wq, K_ext, V_ext, Wo):
    def body(x_ref, wq_ref, k_ref, v_ref, wo_ref, out_ref,
             kv_i8, scl, ss_r, rs_r, ss_l, rs_l, ss_c, rs_c,
             ss_r2, rs_r2, ss_l2, rs_l2, ss_c2, rs_c2):
        my = lax.axis_index("i")
        g = _gray(my)
        even = lax.rem(g, 2) == 0

        def cyc(d):
            return _gray(lax.rem(g + d + 2 * N_DEV, N_DEV))

        def sel(a, b):
            return jnp.where(even, a, b)

        right = cyc(1)
        left = cyc(-1)
        chord = sel(cyc(3), cyc(-3))

        for ref, which in ((k_ref, 0), (v_ref, 1)):
            val = ref[...].reshape(B, S, HD)
            mx = jnp.maximum(
                jnp.max(jnp.abs(val), axis=-1, keepdims=True), 1e-9)
            q = jnp.clip(jnp.round(val * (127.0 / mx)), -127, 127)
            kv_i8[pl.ds(my, 1), :, :, which * HD:(which + 1) * HD] = (
                q.astype(jnp.int8)[None])
            scl[pl.ds(my, 1), which] = mx[..., 0][None]

        barrier_sem = pltpu.get_barrier_semaphore()
        for nbr in (left, right, chord):
            pl.semaphore_signal(
                barrier_sem, inc=1,
                device_id=(nbr,), device_id_type=pl.DeviceIdType.MESH,
            )
        pl.semaphore_wait(barrier_sem, 3)

        def mk_pair(h, chunk, dev, ss, rs, ss2, rs2):
            data = pltpu.make_async_remote_copy(
                src_ref=kv_i8.at[chunk],
                dst_ref=kv_i8.at[chunk],
                send_sem=ss.at[h], recv_sem=rs.at[h],
                device_id=(dev,), device_id_type=pl.DeviceIdType.MESH,
            )
            scale = pltpu.make_async_remote_copy(
                src_ref=scl.at[chunk],
                dst_ref=scl.at[chunk],
                send_sem=ss2.at[h], recv_sem=rs2.at[h],
                device_id=(dev,), device_id_type=pl.DeviceIdType.MESH,
            )
            return (data, scale)

        def start(pair):
            pair[0].start()
            pair[1].start()
            return pair

        def wait(pair):
            pair[0].wait()
            pair[1].wait()

        r_send = [my, cyc(-1)]
        l_send = [my, cyc(1)]
        c_send = [my, sel(cyc(-1), cyc(1)), sel(cyc(-2), cyc(2))]
        r_recv = [cyc(-1), cyc(-2)]
        l_recv = [cyc(1), cyc(2)]
        c_recv = [sel(cyc(3), cyc(-3)), cyc(4), sel(cyc(-3), cyc(3))]

        mk_r = lambda h: mk_pair(h, r_send[h], right, ss_r, rs_r, ss_r2, rs_r2)
        mk_l = lambda h: mk_pair(h, l_send[h], left, ss_l, rs_l, ss_l2, rs_l2)
        mk_c = lambda h: mk_pair(h, c_send[h], chord, ss_c, rs_c, ss_c2, rs_c2)

        rdma_r = [start(mk_r(0))]
        rdma_l = [start(mk_l(0))]
        rdma_c = [start(mk_c(0))]

        wq_bf = wq_ref[...].astype(jnp.bfloat16)
        q_vals = []
        for b in range(B):
            q_b = jnp.dot(x_ref[b].astype(jnp.bfloat16), wq_bf,
                          preferred_element_type=jnp.float32)
            q_vals.append(q_b.astype(jnp.bfloat16))

        iq = lax.broadcasted_iota(jnp.int32, (S, S), 0)
        jk = lax.broadcasted_iota(jnp.int32, (S, S), 1)
        qblk = my * (S // 64) + iq // 64
        kloc = jk // 64

        st = {}

        def process_chunk(c, first):
            kblk = c * (S // 64) + kloc
            msk = (qblk == kblk) | (kblk == 0) | (lax.rem(qblk + kblk, 3) == 0)
            for b in range(B):
                kv_c = kv_i8[pl.ds(c, 1), b]
                k_c = kv_c[0, :, 0:HD].astype(jnp.bfloat16)
                v_c = kv_c[0, :, HD:2 * HD].astype(jnp.bfloat16)
                sk = scl[pl.ds(c, 1), 0, b] * (0.125 / 127.0)
                sv = scl[pl.ds(c, 1), 1, b] * (1.0 / 127.0)
                for h in range(HQ):
                    q_h = q_vals[b][:, h * DH:(h + 1) * DH]
                    k_h = k_c[:, h * DH:(h + 1) * DH]
                    v_h = v_c[:, h * DH:(h + 1) * DH]
                    s_c = lax.dot_general(
                        q_h, k_h,
                        dimension_numbers=(((1,), (1,)), ((), ())),
                        preferred_element_type=jnp.float32,
                    ) * sk
                    s_c = jnp.where(msk, s_c, -1e9)
                    m_c = jnp.max(s_c, axis=1, keepdims=True)
                    if first:
                        p = jnp.exp(s_c - m_c)
                        st[(b, h)] = (
                            m_c,
                            jnp.sum(p, axis=1, keepdims=True),
                            jnp.dot((p * sv).astype(jnp.bfloat16), v_h,
                                    preferred_element_type=jnp.float32),
                        )
                    else:
                        m, l, acc = st[(b, h)]
                        m_new = jnp.maximum(m, m_c)
                        alpha = jnp.exp(m - m_new)
                        p = jnp.exp(s_c - m_new)
                        st[(b, h)] = (
                            m_new,
                            l * alpha + jnp.sum(p, axis=1, keepdims=True),
                            acc * alpha
                            + jnp.dot((p * sv).astype(jnp.bfloat16), v_h,
                                      preferred_element_type=jnp.float32),
                        )

        process_chunk(my, first=True)

        wait(rdma_r[0])
        wait(rdma_l[0])
        wait(rdma_c[0])
        rdma_r.append(start(mk_r(1)))
        rdma_l.append(start(mk_l(1)))
        rdma_c.append(start(mk_c(1)))
        process_chunk(r_recv[0], False)
        process_chunk(l_recv[0], False)
        process_chunk(c_recv[0], False)

        wait(rdma_r[1])
        wait(rdma_l[1])
        wait(rdma_c[1])
        rdma_c.append(start(mk_c(2)))
        process_chunk(r_recv[1], False)
        process_chunk(l_recv[1], False)
        process_chunk(c_recv[1], False)

        wait(rdma_c[2])
        process_chunk(c_recv[2], False)

        for b in range(B):
            ctx_heads = []
            for h in range(HQ):
                m, l, acc = st[(b, h)]
                ctx_heads.append((acc / l).astype(jnp.bfloat16))
            ctx_b = jnp.concatenate(ctx_heads, axis=1)
            out_ref[b] = jnp.dot(ctx_b, wo_ref[...].astype(jnp.bfloat16),
                                 preferred_element_type=jnp.float32)

    return pl.pallas_call(
        body,
        out_shape=jax.ShapeDtypeStruct((B, S, D), jnp.float32),
        in_specs=[pl.BlockSpec(memory_space=pltpu.VMEM)] * 5,
        out_specs=pl.BlockSpec(memory_space=pltpu.VMEM),
        scratch_shapes=[
            pltpu.VMEM((N_DEV, B, S, 2 * HD), jnp.int8),
            pltpu.VMEM((N_DEV, 2, B, S), jnp.float32),
            pltpu.SemaphoreType.DMA((2,)),
            pltpu.SemaphoreType.DMA((2,)),
            pltpu.SemaphoreType.DMA((2,)),
            pltpu.SemaphoreType.DMA((2,)),
            pltpu.SemaphoreType.DMA((3,)),
            pltpu.SemaphoreType.DMA((3,)),
            pltpu.SemaphoreType.DMA((2,)),
            pltpu.SemaphoreType.DMA((2,)),
            pltpu.SemaphoreType.DMA((2,)),
            pltpu.SemaphoreType.DMA((2,)),
            pltpu.SemaphoreType.DMA((3,)),
            pltpu.SemaphoreType.DMA((3,)),
        ],
        compiler_params=pltpu.CompilerParams(collective_id=0),
    )(x, Wq, K_ext, V_ext, Wo)
